# Initial kernel scaffold; baseline (speedup 1.0000x reference)
import jax
import jax.numpy as jnp
from jax.experimental import pallas as pl


def kernel(hidden_states, w_router, ws, w2s):
    raise NotImplementedError("write your pallas kernel here")


# speedup vs baseline: 1.4547x; 1.4547x over previous
import functools

import jax
import jax.numpy as jnp
from jax.experimental import pallas as pl
from jax.experimental.pallas import tpu as pltpu

NUM_EXPERTS = 8
TOP_K = 2
D_MODEL = 1024
D_FF = 2048
FBLK = 512
NF = D_FF // FBLK


def _router_body(x_ref, wr_ref, comb_ref):
    x = x_ref[...]
    wr = wr_ref[...]
    logits = jax.lax.dot_general(
        x, wr, (((1,), (1,)), ((), ())), preferred_element_type=jnp.float32)
    m = jnp.max(logits, axis=1, keepdims=True)
    ex = jnp.exp(logits - m)
    p = ex / jnp.sum(ex, axis=1, keepdims=True)
    lane = jax.lax.broadcasted_iota(jnp.int32, p.shape, 1)
    m0 = jnp.max(p, axis=1, keepdims=True)
    i0 = jnp.min(jnp.where(p == m0, lane, NUM_EXPERTS), axis=1, keepdims=True)
    p1 = jnp.where(lane == i0, -jnp.inf, p)
    m1 = jnp.max(p1, axis=1, keepdims=True)
    i1 = jnp.min(jnp.where(p1 == m1, lane, NUM_EXPERTS), axis=1, keepdims=True)
    s = m0 + m1
    comb_ref[...] = (jnp.where(lane == i0, m0 / s, 0.0)
                     + jnp.where(lane == i1, m1 / s, 0.0))


def _expert_body(comb_ref, x_ref, w1_ref, v1_ref, w2_ref, out_ref):
    e = pl.program_id(0)
    f = pl.program_id(1)
    x = x_ref[...]
    w1 = w1_ref[0]
    v1 = v1_ref[0]
    w2 = w2_ref[0]
    a = jax.lax.dot_general(
        x, w1, (((1,), (1,)), ((), ())), preferred_element_type=jnp.float32)
    b = jax.lax.dot_general(
        x, v1, (((1,), (1,)), ((), ())), preferred_element_type=jnp.float32)
    h = (a * jax.lax.logistic(a)) * b
    y = jax.lax.dot_general(
        h, w2, (((1,), (1,)), ((), ())), preferred_element_type=jnp.float32)
    ids = jax.lax.broadcasted_iota(jnp.int32, (NUM_EXPERTS, 1), 0)
    onehot = (ids == e).astype(jnp.float32)
    c = jax.lax.dot_general(
        comb_ref[...], onehot, (((1,), (0,)), ((), ())),
        preferred_element_type=jnp.float32)

    @pl.when(jnp.logical_and(e == 0, f == 0))
    def _():
        out_ref[...] = jnp.zeros_like(out_ref)

    out_ref[...] += y * c


@functools.partial(jax.jit, static_argnames=("interpret",))
def _run(x, w_router, ws, w2s, interpret=False):
    t = x.shape[0]
    comb = pl.pallas_call(
        _router_body,
        out_shape=jax.ShapeDtypeStruct((t, NUM_EXPERTS), jnp.float32),
        interpret=interpret,
    )(x, w_router)

    out = pl.pallas_call(
        _expert_body,
        grid=(NUM_EXPERTS, NF),
        in_specs=[
            pl.BlockSpec((t, NUM_EXPERTS), lambda e, f: (0, 0)),
            pl.BlockSpec((t, D_MODEL), lambda e, f: (0, 0)),
            pl.BlockSpec((1, FBLK, D_MODEL), lambda e, f: (e, f, 0)),
            pl.BlockSpec((1, FBLK, D_MODEL), lambda e, f: (e, NF + f, 0)),
            pl.BlockSpec((1, D_MODEL, FBLK), lambda e, f: (e, 0, f)),
        ],
        out_specs=pl.BlockSpec((t, D_MODEL), lambda e, f: (0, 0)),
        out_shape=jax.ShapeDtypeStruct((t, D_MODEL), jnp.float32),
        compiler_params=pltpu.CompilerParams(
            dimension_semantics=("arbitrary", "arbitrary")),
        interpret=interpret,
    )(comb, x, ws, ws, w2s)
    return out


def kernel(hidden_states, w_router, ws, w2s):
    b, s, d = hidden_states.shape
    x = hidden_states.reshape(s, d)
    out = _run(x, w_router, ws, w2s)
    return out.reshape(b, s, d)


# speedup vs baseline: 1.4610x; 1.0043x over previous
import functools

import jax
import jax.numpy as jnp
from jax.experimental import pallas as pl
from jax.experimental.pallas import tpu as pltpu

NUM_EXPERTS = 8
TOP_K = 2
D_MODEL = 1024
D_FF = 2048
FBLK = 512
NF = D_FF // FBLK


def _router_body(x_ref, wr_ref, comb_ref):
    x = x_ref[...]
    wr = wr_ref[...]
    logits = jax.lax.dot_general(
        x, wr, (((1,), (1,)), ((), ())), preferred_element_type=jnp.float32)
    m = jnp.max(logits, axis=1, keepdims=True)
    ex = jnp.exp(logits - m)
    p = ex / jnp.sum(ex, axis=1, keepdims=True)
    lane = jax.lax.broadcasted_iota(jnp.int32, p.shape, 1)
    m0 = jnp.max(p, axis=1, keepdims=True)
    i0 = jnp.min(jnp.where(p == m0, lane, NUM_EXPERTS), axis=1, keepdims=True)
    p1 = jnp.where(lane == i0, -jnp.inf, p)
    m1 = jnp.max(p1, axis=1, keepdims=True)
    i1 = jnp.min(jnp.where(p1 == m1, lane, NUM_EXPERTS), axis=1, keepdims=True)
    s = m0 + m1
    comb_ref[...] = (jnp.where(lane == i0, m0 / s, 0.0)
                     + jnp.where(lane == i1, m1 / s, 0.0))


def _expert_body(comb_ref, x_ref, w1_ref, v1_ref, w2_ref, out_ref):
    e = pl.program_id(0)
    f = pl.program_id(1)
    x = x_ref[...].astype(jnp.bfloat16)
    w1 = w1_ref[0].astype(jnp.bfloat16)
    v1 = v1_ref[0].astype(jnp.bfloat16)
    w2 = w2_ref[0].astype(jnp.bfloat16)
    a = jax.lax.dot_general(
        x, w1, (((1,), (1,)), ((), ())), preferred_element_type=jnp.float32)
    b = jax.lax.dot_general(
        x, v1, (((1,), (1,)), ((), ())), preferred_element_type=jnp.float32)
    h = ((a * jax.lax.logistic(a)) * b).astype(jnp.bfloat16)
    y = jax.lax.dot_general(
        h, w2, (((1,), (1,)), ((), ())), preferred_element_type=jnp.float32)
    ids = jax.lax.broadcasted_iota(jnp.int32, (NUM_EXPERTS, 1), 0)
    onehot = (ids == e).astype(jnp.float32)
    c = jax.lax.dot_general(
        comb_ref[...], onehot, (((1,), (0,)), ((), ())),
        preferred_element_type=jnp.float32)

    @pl.when(jnp.logical_and(e == 0, f == 0))
    def _():
        out_ref[...] = jnp.zeros_like(out_ref)

    out_ref[...] += y * c


@functools.partial(jax.jit, static_argnames=("interpret",))
def _run(x, w_router, ws, w2s, interpret=False):
    t = x.shape[0]
    comb = pl.pallas_call(
        _router_body,
        out_shape=jax.ShapeDtypeStruct((t, NUM_EXPERTS), jnp.float32),
        interpret=interpret,
    )(x, w_router)

    out = pl.pallas_call(
        _expert_body,
        grid=(NUM_EXPERTS, NF),
        in_specs=[
            pl.BlockSpec((t, NUM_EXPERTS), lambda e, f: (0, 0)),
            pl.BlockSpec((t, D_MODEL), lambda e, f: (0, 0)),
            pl.BlockSpec((1, FBLK, D_MODEL), lambda e, f: (e, f, 0)),
            pl.BlockSpec((1, FBLK, D_MODEL), lambda e, f: (e, NF + f, 0)),
            pl.BlockSpec((1, D_MODEL, FBLK), lambda e, f: (e, 0, f)),
        ],
        out_specs=pl.BlockSpec((t, D_MODEL), lambda e, f: (0, 0)),
        out_shape=jax.ShapeDtypeStruct((t, D_MODEL), jnp.float32),
        compiler_params=pltpu.CompilerParams(
            dimension_semantics=("arbitrary", "arbitrary")),
        interpret=interpret,
    )(comb, x, ws, ws, w2s)
    return out


def kernel(hidden_states, w_router, ws, w2s):
    b, s, d = hidden_states.shape
    x = hidden_states.reshape(s, d)
    out = _run(x, w_router, ws, w2s)
    return out.reshape(b, s, d)
